# Initial kernel scaffold; baseline (speedup 1.0000x reference)
import jax
import jax.numpy as jnp
from jax.experimental import pallas as pl


def kernel(x, norm_w, router_w, w1, w2, w3, gamma):
    raise NotImplementedError("write your pallas kernel here")


# speedup vs baseline: 1.2488x; 1.2488x over previous
import functools

import jax
import jax.numpy as jnp
from jax.experimental import pallas as pl
from jax.experimental.pallas import tpu as pltpu

DIM = 2048
HID = 4 * DIM
N_TOK = 2048
THRESH = 0.35
EPS = 1e-6
BH = 256
NJ = HID // BH


def _fused_block_kernel(x_ref, nw_ref, rw_ref, w1_ref, w2_ref, w3_ref,
                        gamma_ref, out_ref, xn_ref, m_ref):
    j = pl.program_id(0)

    @pl.when(j == 0)
    def _prologue():
        xf = x_ref[...]
        ms = jnp.mean(xf * xf, axis=-1, keepdims=True)
        xn = xf * jax.lax.rsqrt(ms + EPS) * nw_ref[...]
        g = jnp.sum(xn * rw_ref[...], axis=-1, keepdims=True)
        act = (jax.nn.sigmoid(g) > THRESH).astype(jnp.float32)
        xn_ref[...] = xn.astype(jnp.bfloat16)
        m_ref[...] = act
        out_ref[...] = jnp.zeros_like(out_ref)

    xn = xn_ref[...]
    w1b = w1_ref[...].astype(jnp.bfloat16)
    w3b = w3_ref[...].astype(jnp.bfloat16)
    w2b = w2_ref[...].astype(jnp.bfloat16)
    u = jax.lax.dot_general(xn, w1b, (((1,), (1,)), ((), ())),
                            preferred_element_type=jnp.float32)
    v = jax.lax.dot_general(xn, w3b, (((1,), (1,)), ((), ())),
                            preferred_element_type=jnp.float32)
    h = (u * jax.nn.sigmoid(u) * v).astype(jnp.bfloat16)
    t = jax.lax.dot_general(h, w2b, (((1,), (1,)), ((), ())),
                            preferred_element_type=jnp.float32)
    out_ref[...] += t

    @pl.when(j == NJ - 1)
    def _epilogue():
        out_ref[...] = x_ref[...] + out_ref[...] * (m_ref[...] * gamma_ref[...])


@jax.jit
def kernel(x, norm_w, router_w, w1, w2, w3, gamma):
    nw = norm_w.reshape(1, DIM)
    gm = gamma.reshape(1, DIM)
    out = pl.pallas_call(
        _fused_block_kernel,
        grid=(NJ,),
        in_specs=[
            pl.BlockSpec((N_TOK, DIM), lambda j: (0, 0)),
            pl.BlockSpec((1, DIM), lambda j: (0, 0)),
            pl.BlockSpec((1, DIM), lambda j: (0, 0)),
            pl.BlockSpec((BH, DIM), lambda j: (j, 0)),
            pl.BlockSpec((DIM, BH), lambda j: (0, j)),
            pl.BlockSpec((BH, DIM), lambda j: (j, 0)),
            pl.BlockSpec((1, DIM), lambda j: (0, 0)),
        ],
        out_specs=pl.BlockSpec((N_TOK, DIM), lambda j: (0, 0)),
        out_shape=jax.ShapeDtypeStruct((N_TOK, DIM), jnp.float32),
        scratch_shapes=[
            pltpu.VMEM((N_TOK, DIM), jnp.bfloat16),
            pltpu.VMEM((N_TOK, 1), jnp.float32),
        ],
    )(x, nw, router_w, w1, w2, w3, gm)
    return out
